# baseline (speedup 1.0000x reference)
import functools
import math

import jax
import jax.numpy as jnp
import numpy as np
from jax import lax
from jax.experimental import pallas as pl
from jax.experimental.pallas import tpu as pltpu
from jax.experimental.pallas import tpu_sc as plsc

VOCAB = 100000
D_MODEL = 768
MAX_SEQ_LEN = 2048
BATCH = 4
SEQ_LEN = 2048

NUM_WORKERS = 32
TOK_PER_W = (BATCH * SEQ_LEN) // NUM_WORKERS
CHUNK = 32
NCHUNK = TOK_PER_W // CHUNK
LANES = 16
GROUPS = D_MODEL // LANES
SCALE = math.sqrt(D_MODEL)


def _make_pe_np(max_seq_len, d_model):
    pe = np.zeros((max_seq_len, d_model), dtype=np.float32)
    position = np.arange(0, max_seq_len, dtype=np.float32)[:, None]
    div_term = np.exp(
        np.arange(0, d_model, 2, dtype=np.float32) * (-math.log(10000.0) / d_model)
    )
    pe[:, 0::2] = np.sin(position * div_term)
    pe[:, 1::2] = np.cos(position * div_term)
    return pe


_PE = _make_pe_np(MAX_SEQ_LEN, D_MODEL)

_SEQ_PER_W = SEQ_LEN // (NUM_WORKERS // BATCH)


def _body(ids_hbm, pe_hbm, table_hbm, out_hbm, idx_v, rows_v, pe_v,
          gsem, psem, osem):
    wid = lax.axis_index("s") * 2 + lax.axis_index("c")
    base = wid * TOK_PER_W
    s0 = lax.rem(wid, NUM_WORKERS // BATCH) * _SEQ_PER_W

    pltpu.sync_copy(ids_hbm.at[wid], idx_v)

    def gather(c, slot):
        g = pltpu.async_copy(table_hbm.at[idx_v.at[c]], rows_v.at[slot],
                             gsem.at[slot])
        p = pltpu.async_copy(pe_hbm.at[pl.ds(s0 + c * CHUNK, CHUNK)],
                             pe_v.at[slot], psem.at[slot])
        return g, p

    def put(c, slot):
        return pltpu.async_copy(rows_v.at[slot],
                                out_hbm.at[pl.ds(base + c * CHUNK, CHUNK)],
                                osem.at[slot])

    def compute(slot):
        def row_body(t, carry):
            for j in range(GROUPS):
                sl = pl.ds(j * LANES, LANES)
                r = rows_v[slot, t, sl]
                p = pe_v[slot, t, sl]
                rows_v[slot, t, sl] = r * SCALE + p
            return carry
        lax.fori_loop(0, CHUNK, row_body, 0)

    pending_g = {}
    pending_o = {}
    pending_g[0] = gather(0, 0)
    for c in range(NCHUNK):
        slot = c % 2
        nslot = 1 - slot
        if c + 1 < NCHUNK:
            if c - 1 >= 0:
                pending_o.pop(c - 1).wait()
            pending_g[c + 1] = gather(c + 1, nslot)
        g, p = pending_g.pop(c)
        g.wait()
        p.wait()
        compute(slot)
        pending_o[c] = put(c, slot)
    pending_o.pop(NCHUNK - 2).wait()
    pending_o.pop(NCHUNK - 1).wait()


@jax.jit
def _combined_embedding(ids3, pe, W):
    mesh = plsc.VectorSubcoreMesh(core_axis_name="c", subcore_axis_name="s",
                                  num_cores=2, num_subcores=16)
    return pl.kernel(
        _body,
        out_type=jax.ShapeDtypeStruct((BATCH * SEQ_LEN, D_MODEL), jnp.float32),
        mesh=mesh,
        scratch_types=[
            pltpu.VMEM((NCHUNK, CHUNK), jnp.int32),
            pltpu.VMEM((2, CHUNK, D_MODEL), jnp.float32),
            pltpu.VMEM((2, CHUNK, D_MODEL), jnp.float32),
            pltpu.SemaphoreType.DMA((2,)),
            pltpu.SemaphoreType.DMA((2,)),
            pltpu.SemaphoreType.DMA((2,)),
        ],
    )(ids3, pe, W)


def kernel(token_ids, W):
    ids3 = token_ids.astype(jnp.int32).reshape(NUM_WORKERS, NCHUNK, CHUNK)
    out = _combined_embedding(ids3, _PE, W)
    return out.reshape(BATCH, SEQ_LEN, D_MODEL)


# speedup vs baseline: 1.0678x; 1.0678x over previous
import functools
import math

import jax
import jax.numpy as jnp
import numpy as np
from jax import lax
from jax.experimental import pallas as pl
from jax.experimental.pallas import tpu as pltpu
from jax.experimental.pallas import tpu_sc as plsc

VOCAB = 100000
D_MODEL = 768
MAX_SEQ_LEN = 2048
BATCH = 4
SEQ_LEN = 2048

NUM_WORKERS = 32
POS_PER_W = SEQ_LEN // NUM_WORKERS
TOK_PER_W = BATCH * POS_PER_W
CHUNK = 32
NCHUNK = TOK_PER_W // CHUNK
HALVES = POS_PER_W // CHUNK
LANES = 16
GROUPS = D_MODEL // LANES
SCALE = math.sqrt(D_MODEL)


def _make_pe_np(max_seq_len, d_model):
    pe = np.zeros((max_seq_len, d_model), dtype=np.float32)
    position = np.arange(0, max_seq_len, dtype=np.float32)[:, None]
    div_term = np.exp(
        np.arange(0, d_model, 2, dtype=np.float32) * (-math.log(10000.0) / d_model)
    )
    pe[:, 0::2] = np.sin(position * div_term)
    pe[:, 1::2] = np.cos(position * div_term)
    return pe


_PE = _make_pe_np(MAX_SEQ_LEN, D_MODEL)


def _body(ids_hbm, pe_hbm, table_hbm, out_hbm, idx_v, rows_v, pe_v,
          gsem, osem, psem):
    wid = lax.axis_index("s") * 2 + lax.axis_index("c")
    s0 = wid * POS_PER_W

    pe_dma = pltpu.async_copy(pe_hbm.at[pl.ds(s0, POS_PER_W)], pe_v, psem)
    pltpu.sync_copy(ids_hbm.at[wid], idx_v)

    def gather(c, slot):
        return pltpu.async_copy(table_hbm.at[idx_v.at[c]], rows_v.at[slot],
                                gsem.at[slot])

    def put(c, slot):
        b = c // HALVES
        off = b * SEQ_LEN + s0 + (c % HALVES) * CHUNK
        return pltpu.async_copy(rows_v.at[slot],
                                out_hbm.at[pl.ds(off, CHUNK)],
                                osem.at[slot])

    def compute(c, slot):
        pbase = (c % HALVES) * CHUNK

        def row_body(t, carry):
            for j in range(GROUPS):
                sl = pl.ds(j * LANES, LANES)
                r = rows_v[slot, t, sl]
                p = pe_v[pbase + t, sl]
                rows_v[slot, t, sl] = r * SCALE + p
            return carry

        lax.fori_loop(0, CHUNK, row_body, 0, unroll=False)

    pending_g = {}
    pending_o = {}
    pending_g[0] = gather(0, 0)
    pe_dma.wait()
    for c in range(NCHUNK):
        slot = c % 2
        if c + 1 < NCHUNK:
            if c - 1 >= 0:
                pending_o.pop(c - 1).wait()
            pending_g[c + 1] = gather(c + 1, 1 - slot)
        pending_g.pop(c).wait()
        compute(c, slot)
        pending_o[c] = put(c, slot)
    pending_o.pop(NCHUNK - 2).wait()
    pending_o.pop(NCHUNK - 1).wait()


@jax.jit
def _combined_embedding(ids3, pe, W):
    mesh = plsc.VectorSubcoreMesh(core_axis_name="c", subcore_axis_name="s",
                                  num_cores=2, num_subcores=16)
    return pl.kernel(
        _body,
        out_type=jax.ShapeDtypeStruct((BATCH * SEQ_LEN, D_MODEL), jnp.float32),
        mesh=mesh,
        scratch_types=[
            pltpu.VMEM((NCHUNK, CHUNK), jnp.int32),
            pltpu.VMEM((2, CHUNK, D_MODEL), jnp.float32),
            pltpu.VMEM((POS_PER_W, D_MODEL), jnp.float32),
            pltpu.SemaphoreType.DMA((2,)),
            pltpu.SemaphoreType.DMA((2,)),
            pltpu.SemaphoreType.DMA,
        ],
    )(ids3, pe, W)


def kernel(token_ids, W):
    ids = token_ids.astype(jnp.int32).reshape(BATCH, NUM_WORKERS, HALVES, CHUNK)
    ids3 = ids.transpose(1, 0, 2, 3).reshape(NUM_WORKERS, NCHUNK, CHUNK)
    out = _combined_embedding(ids3, _PE, W)
    return out.reshape(BATCH, SEQ_LEN, D_MODEL)


# speedup vs baseline: 1.4446x; 1.3529x over previous
import functools
import math

import jax
import jax.numpy as jnp
import numpy as np
from jax import lax
from jax.experimental import pallas as pl
from jax.experimental.pallas import tpu as pltpu
from jax.experimental.pallas import tpu_sc as plsc

VOCAB = 100000
D_MODEL = 768
MAX_SEQ_LEN = 2048
BATCH = 4
SEQ_LEN = 2048

NUM_WORKERS = 32
POS_PER_W = SEQ_LEN // NUM_WORKERS
TOK_PER_W = BATCH * POS_PER_W
CHUNK = 32
NCHUNK = TOK_PER_W // CHUNK
HALVES = POS_PER_W // CHUNK
LANES = 16
GROUPS = D_MODEL // LANES
SCALE = math.sqrt(D_MODEL)


def _make_pe_np(max_seq_len, d_model):
    pe = np.zeros((max_seq_len, d_model), dtype=np.float32)
    position = np.arange(0, max_seq_len, dtype=np.float32)[:, None]
    div_term = np.exp(
        np.arange(0, d_model, 2, dtype=np.float32) * (-math.log(10000.0) / d_model)
    )
    pe[:, 0::2] = np.sin(position * div_term)
    pe[:, 1::2] = np.cos(position * div_term)
    return pe


_PE = _make_pe_np(MAX_SEQ_LEN, D_MODEL)


def _body(ids_hbm, pe_hbm, table_hbm, out_hbm, idx_v, rows_v, pe_v,
          gsem, osem, psem):
    wid = lax.axis_index("s") * 2 + lax.axis_index("c")
    s0 = wid * POS_PER_W

    pe_dma = pltpu.async_copy(pe_hbm.at[pl.ds(s0, POS_PER_W)], pe_v, psem)
    pltpu.sync_copy(ids_hbm.at[wid], idx_v)

    def gather(c, slot):
        return pltpu.async_copy(table_hbm.at[idx_v.at[c]], rows_v.at[slot],
                                gsem.at[slot])

    def put(c, slot):
        b = c // HALVES
        off = b * SEQ_LEN + s0 + (c % HALVES) * CHUNK
        return pltpu.async_copy(rows_v.at[slot],
                                out_hbm.at[pl.ds(off, CHUNK)],
                                osem.at[slot])

    def compute(c, slot):
        pbase = (c % HALVES) * CHUNK

        @plsc.parallel_loop(0, CHUNK, step=1)
        def _row_body(t):
            for j in range(GROUPS):
                sl = pl.ds(j * LANES, LANES)
                r = rows_v[slot, t, sl]
                p = pe_v[pbase + t, sl]
                rows_v[slot, t, sl] = r * SCALE + p

    pending_g = {}
    pending_o = {}
    pending_g[0] = gather(0, 0)
    pe_dma.wait()
    for c in range(NCHUNK):
        slot = c % 2
        if c + 1 < NCHUNK:
            if c - 1 >= 0:
                pending_o.pop(c - 1).wait()
            pending_g[c + 1] = gather(c + 1, 1 - slot)
        pending_g.pop(c).wait()
        compute(c, slot)
        pending_o[c] = put(c, slot)
    pending_o.pop(NCHUNK - 2).wait()
    pending_o.pop(NCHUNK - 1).wait()


@jax.jit
def _combined_embedding(ids3, pe, W):
    mesh = plsc.VectorSubcoreMesh(core_axis_name="c", subcore_axis_name="s",
                                  num_cores=2, num_subcores=16)
    return pl.kernel(
        _body,
        out_type=jax.ShapeDtypeStruct((BATCH * SEQ_LEN, D_MODEL), jnp.float32),
        mesh=mesh,
        scratch_types=[
            pltpu.VMEM((NCHUNK, CHUNK), jnp.int32),
            pltpu.VMEM((2, CHUNK, D_MODEL), jnp.float32),
            pltpu.VMEM((POS_PER_W, D_MODEL), jnp.float32),
            pltpu.SemaphoreType.DMA((2,)),
            pltpu.SemaphoreType.DMA((2,)),
            pltpu.SemaphoreType.DMA,
        ],
    )(ids3, pe, W)


def kernel(token_ids, W):
    ids = token_ids.astype(jnp.int32).reshape(BATCH, NUM_WORKERS, HALVES, CHUNK)
    ids3 = ids.transpose(1, 0, 2, 3).reshape(NUM_WORKERS, NCHUNK, CHUNK)
    out = _combined_embedding(ids3, _PE, W)
    return out.reshape(BATCH, SEQ_LEN, D_MODEL)
